# baseline (speedup 1.0000x reference)
import jax
import jax.numpy as jnp
from jax.experimental import pallas as pl

_BETA = 0.25
_TB = 8


def _pq_step(zn_ref, ze_ref, cb_ref, zq_ref, tok_ref, loss_ref, util_ref):
    t = pl.program_id(0)
    B = ze_ref.shape[0]
    K = cb_ref.shape[1]

    @pl.when(t == 0)
    def _init():
        loss_ref[:, :] = jnp.zeros((1, 1), jnp.float32)
        util_ref[:, :] = jnp.zeros((1, 1), jnp.float32)

    loss_acc = jnp.zeros((), jnp.float32)
    util_acc = jnp.zeros((), jnp.float32)
    for i in range(_TB):
        ze = ze_ref[:, i, :]
        cb = cb_ref[i]
        zn = zn_ref[i, 0, :]
        scores = jax.lax.dot_general(
            ze, cb, dimension_numbers=(((1,), (1,)), ((), ())))
        cbn = jnp.sum(cb * cb, axis=-1)
        dist = (zn[:, None] - 2.0 * scores) + cbn[None, :]
        m = jnp.min(dist, axis=-1, keepdims=True)
        iota_k = jax.lax.broadcasted_iota(jnp.int32, (B, K), 1)
        k_idx = jnp.min(jnp.where(dist == m, iota_k, K), axis=-1).astype(jnp.int32)
        onehot = (k_idx[:, None] == iota_k).astype(jnp.float32)
        zq = jax.lax.dot_general(
            onehot, cb, dimension_numbers=(((1,), (0,)), ((), ())),
            precision=jax.lax.Precision.HIGHEST)
        zq_ref[:, i, :] = ze + (zq - ze)
        tok_ref[i, 0, :] = k_idx
        d = ze - zq
        loss_acc = loss_acc + jnp.sum(d * d)
        util_acc = util_acc + jnp.sum(jnp.max(onehot, axis=0))

    loss_ref[:, :] = loss_ref[:, :] + loss_acc
    util_ref[:, :] = util_ref[:, :] + util_acc


def kernel(z_e, codebooks):
    B, T, D = z_e.shape
    K = codebooks.shape[1]
    zn_t = jnp.sum(z_e ** 2, axis=-1).T.reshape(T, 1, B)
    z_q_st, tok_t, loss, util = pl.pallas_call(
        _pq_step,
        grid=(T // _TB,),
        in_specs=[
            pl.BlockSpec((_TB, 1, B), lambda t: (t, 0, 0)),
            pl.BlockSpec((B, _TB, D), lambda t: (0, t, 0)),
            pl.BlockSpec((_TB, K, D), lambda t: (t, 0, 0)),
        ],
        out_specs=[
            pl.BlockSpec((B, _TB, D), lambda t: (0, t, 0)),
            pl.BlockSpec((_TB, 1, B), lambda t: (t, 0, 0)),
            pl.BlockSpec((1, 1), lambda t: (0, 0)),
            pl.BlockSpec((1, 1), lambda t: (0, 0)),
        ],
        out_shape=[
            jax.ShapeDtypeStruct((B, T, D), jnp.float32),
            jax.ShapeDtypeStruct((T, 1, B), jnp.int32),
            jax.ShapeDtypeStruct((1, 1), jnp.float32),
            jax.ShapeDtypeStruct((1, 1), jnp.float32),
        ],
    )(zn_t, z_e, codebooks)
    tokens = jnp.transpose(tok_t[:, 0, :], (1, 0))
    vq_loss = _BETA * (loss[0, 0] / jnp.float32(T * B * D))
    utilization = util[0, 0] / jnp.float32(T * K)
    return z_q_st, tokens, vq_loss, utilization


# speedup vs baseline: 1.0210x; 1.0210x over previous
import jax
import jax.numpy as jnp
from jax.experimental import pallas as pl

_BETA = 0.25
_TB = 32


def _pq_step(zn_ref, ze_ref, cb_ref, zq_ref, tok_ref, loss_ref, util_ref):
    t = pl.program_id(0)
    B = ze_ref.shape[1]
    K = cb_ref.shape[1]

    @pl.when(t == 0)
    def _init():
        loss_ref[:, :] = jnp.zeros((1, 1), jnp.float32)
        util_ref[:, :] = jnp.zeros((1, 1), jnp.float32)

    loss_acc = jnp.zeros((), jnp.float32)
    util_acc = jnp.zeros((), jnp.float32)
    for i in range(_TB):
        ze = ze_ref[i]
        cb = cb_ref[i]
        zn = zn_ref[i, 0, :]
        scores = jax.lax.dot_general(
            ze, cb, dimension_numbers=(((1,), (1,)), ((), ())))
        cbn = jnp.sum(cb * cb, axis=-1)
        dist = (zn[:, None] - 2.0 * scores) + cbn[None, :]
        m = jnp.min(dist, axis=-1, keepdims=True)
        iota_k = jax.lax.broadcasted_iota(jnp.int32, (B, K), 1)
        k_idx = jnp.min(jnp.where(dist == m, iota_k, K), axis=-1).astype(jnp.int32)
        onehot = (k_idx[:, None] == iota_k).astype(jnp.float32)
        zq = jax.lax.dot_general(
            onehot, cb, dimension_numbers=(((1,), (0,)), ((), ())),
            precision=jax.lax.Precision.HIGHEST)
        zq_ref[i] = ze + (zq - ze)
        tok_ref[i, 0, :] = k_idx
        d = ze - zq
        loss_acc = loss_acc + jnp.sum(d * d)
        util_acc = util_acc + jnp.sum(jnp.max(onehot, axis=0))

    loss_ref[:, :] = loss_ref[:, :] + loss_acc
    util_ref[:, :] = util_ref[:, :] + util_acc


def kernel(z_e, codebooks):
    B, T, D = z_e.shape
    K = codebooks.shape[1]
    ze_t = jnp.transpose(z_e, (1, 0, 2))
    zn_t = jnp.sum(ze_t ** 2, axis=-1).reshape(T, 1, B)
    zq_t, tok_t, loss, util = pl.pallas_call(
        _pq_step,
        grid=(T // _TB,),
        in_specs=[
            pl.BlockSpec((_TB, 1, B), lambda t: (t, 0, 0)),
            pl.BlockSpec((_TB, B, D), lambda t: (t, 0, 0)),
            pl.BlockSpec((_TB, K, D), lambda t: (t, 0, 0)),
        ],
        out_specs=[
            pl.BlockSpec((_TB, B, D), lambda t: (t, 0, 0)),
            pl.BlockSpec((_TB, 1, B), lambda t: (t, 0, 0)),
            pl.BlockSpec((1, 1), lambda t: (0, 0)),
            pl.BlockSpec((1, 1), lambda t: (0, 0)),
        ],
        out_shape=[
            jax.ShapeDtypeStruct((T, B, D), jnp.float32),
            jax.ShapeDtypeStruct((T, 1, B), jnp.int32),
            jax.ShapeDtypeStruct((1, 1), jnp.float32),
            jax.ShapeDtypeStruct((1, 1), jnp.float32),
        ],
    )(zn_t, ze_t, codebooks)
    z_q_st = jnp.transpose(zq_t, (1, 0, 2))
    tokens = jnp.transpose(tok_t[:, 0, :], (1, 0))
    vq_loss = _BETA * (loss[0, 0] / jnp.float32(T * B * D))
    utilization = util[0, 0] / jnp.float32(T * K)
    return z_q_st, tokens, vq_loss, utilization


# speedup vs baseline: 1.3503x; 1.3225x over previous
import jax
import jax.numpy as jnp
from jax.experimental import pallas as pl

_BETA = 0.25
_TB = 8


def _pq_step(zn_ref, ze_ref, cb_ref, zq_ref, tok_ref, loss_ref, util_ref):
    t = pl.program_id(0)
    TB, B, D = ze_ref.shape
    K = cb_ref.shape[1]

    @pl.when(t == 0)
    def _init():
        loss_ref[:, :] = jnp.zeros((1, 1), jnp.float32)
        util_ref[:, :] = jnp.zeros((1, 1), jnp.float32)

    ze_all = ze_ref[:]
    cb_all = cb_ref[:]
    zn_all = zn_ref[:, 0, :]

    scores = jnp.stack([
        jax.lax.dot_general(ze_all[i], cb_all[i],
                            dimension_numbers=(((1,), (1,)), ((), ())))
        for i in range(TB)
    ])

    cbn = jnp.sum(cb_all * cb_all, axis=-1)
    dist = (zn_all[:, :, None] - 2.0 * scores) + cbn[:, None, :]
    m = jnp.min(dist, axis=-1, keepdims=True)
    iota_k = jax.lax.broadcasted_iota(jnp.int32, (TB, B, K), 2)
    k_idx = jnp.min(jnp.where(dist == m, iota_k, K), axis=-1).astype(jnp.int32)
    onehot = (k_idx[:, :, None] == iota_k).astype(jnp.float32)

    zq = jnp.stack([
        jax.lax.dot_general(onehot[i], cb_all[i],
                            dimension_numbers=(((1,), (0,)), ((), ())),
                            precision=jax.lax.Precision.HIGHEST)
        for i in range(TB)
    ])

    zq_ref[:] = ze_all + (zq - ze_all)
    tok_ref[:, 0, :] = k_idx
    d = ze_all - zq
    loss_ref[:, :] = loss_ref[:, :] + jnp.sum(d * d)
    util_ref[:, :] = util_ref[:, :] + jnp.sum(jnp.max(onehot, axis=1))


def kernel(z_e, codebooks):
    B, T, D = z_e.shape
    K = codebooks.shape[1]
    ze_t = jnp.transpose(z_e, (1, 0, 2))
    zn_t = jnp.sum(ze_t ** 2, axis=-1).reshape(T, 1, B)
    zq_t, tok_t, loss, util = pl.pallas_call(
        _pq_step,
        grid=(T // _TB,),
        in_specs=[
            pl.BlockSpec((_TB, 1, B), lambda t: (t, 0, 0)),
            pl.BlockSpec((_TB, B, D), lambda t: (t, 0, 0)),
            pl.BlockSpec((_TB, K, D), lambda t: (t, 0, 0)),
        ],
        out_specs=[
            pl.BlockSpec((_TB, B, D), lambda t: (t, 0, 0)),
            pl.BlockSpec((_TB, 1, B), lambda t: (t, 0, 0)),
            pl.BlockSpec((1, 1), lambda t: (0, 0)),
            pl.BlockSpec((1, 1), lambda t: (0, 0)),
        ],
        out_shape=[
            jax.ShapeDtypeStruct((T, B, D), jnp.float32),
            jax.ShapeDtypeStruct((T, 1, B), jnp.int32),
            jax.ShapeDtypeStruct((1, 1), jnp.float32),
            jax.ShapeDtypeStruct((1, 1), jnp.float32),
        ],
    )(zn_t, ze_t, codebooks)
    z_q_st = jnp.transpose(zq_t, (1, 0, 2))
    tokens = jnp.transpose(tok_t[:, 0, :], (1, 0))
    vq_loss = _BETA * (loss[0, 0] / jnp.float32(T * B * D))
    utilization = util[0, 0] / jnp.float32(T * K)
    return z_q_st, tokens, vq_loss, utilization
